# Initial kernel scaffold; baseline (speedup 1.0000x reference)
import jax
import jax.numpy as jnp
from jax.experimental import pallas as pl


def kernel(x, Wg, W1, b1, W2, b2):
    raise NotImplementedError("write your pallas kernel here")


# speedup vs baseline: 2.3826x; 2.3826x over previous
import jax
import jax.numpy as jnp
from jax.experimental import pallas as pl
from jax.experimental.pallas import tpu as pltpu

E = 8
K = 2
D = 768
DFF = 768
OUT = 768
T = 4096
CAP = 1280
COEF = 0.01

TB = 512
NB = T // TB
MB = 256


def _gate_body(x_ref, wg_ref,
               s0_ref, s1_ref, v0_ref, v1_ref, g0_ref, g1_ref, aux_ref,
               imp_ref, carry_ref):
    pid = pl.program_id(0)

    @pl.when(pid == 0)
    def _init():
        imp_ref[...] = jnp.zeros((1, E), jnp.float32)
        carry_ref[...] = jnp.zeros((1, E), jnp.float32)

    x = x_ref[...]
    wg = wg_ref[...]
    logits = jnp.dot(x, wg, preferred_element_type=jnp.float32)

    iota = jax.lax.broadcasted_iota(jnp.int32, (TB, E), 1)
    m0 = jnp.max(logits, axis=1, keepdims=True)
    i0 = jnp.min(jnp.where(logits == m0, iota, E), axis=1, keepdims=True)
    masked = jnp.where(iota == i0, -jnp.inf, logits)
    m1 = jnp.max(masked, axis=1, keepdims=True)
    i1 = jnp.min(jnp.where(masked == m1, iota, E), axis=1, keepdims=True)

    g0 = 1.0 / (1.0 + jnp.exp(m1 - m0))
    g1 = 1.0 / (1.0 + jnp.exp(m0 - m1))

    ohA = (iota == i0).astype(jnp.float32)
    ohB = (iota == i1).astype(jnp.float32)

    imp_ref[...] += jnp.sum(ohA * g0 + ohB * g1, axis=0, keepdims=True)

    r = jax.lax.broadcasted_iota(jnp.int32, (TB, TB), 0)
    c = jax.lax.broadcasted_iota(jnp.int32, (TB, TB), 1)
    lt = (c < r).astype(jnp.float32)
    ab = ohA + ohB
    S = jnp.dot(lt, ab, preferred_element_type=jnp.float32) + carry_ref[...]
    pA = jnp.sum(S * ohA, axis=1, keepdims=True)
    pB = jnp.sum((S + ohA) * ohB, axis=1, keepdims=True)
    carry_ref[...] += jnp.sum(ab, axis=0, keepdims=True)

    kA = pA < CAP
    kB = pB < CAP
    s0_ref[...] = i0 * CAP + jnp.where(kA, pA.astype(jnp.int32), 0)
    s1_ref[...] = i1 * CAP + jnp.where(kB, pB.astype(jnp.int32), 0)
    tok = pid * TB + jax.lax.broadcasted_iota(jnp.int32, (TB, 1), 0)
    v0_ref[...] = jnp.where(kA, tok, -1)
    v1_ref[...] = jnp.where(kB, tok, -1)
    g0_ref[...] = jnp.where(kA, g0, 0.0)
    g1_ref[...] = jnp.where(kB, g1, 0.0)

    @pl.when(pid == NB - 1)
    def _fin():
        imp = imp_ref[...]
        mean = jnp.sum(imp) / E
        var = jnp.sum((imp - mean) ** 2) / E
        aux_ref[...] = jnp.full((1, 1), COEF * var / (mean * mean + 1e-10),
                                jnp.float32)


def _gating(x, Wg):
    out_shapes = (
        jax.ShapeDtypeStruct((T, 1), jnp.int32),
        jax.ShapeDtypeStruct((T, 1), jnp.int32),
        jax.ShapeDtypeStruct((T, 1), jnp.int32),
        jax.ShapeDtypeStruct((T, 1), jnp.int32),
        jax.ShapeDtypeStruct((T, 1), jnp.float32),
        jax.ShapeDtypeStruct((T, 1), jnp.float32),
        jax.ShapeDtypeStruct((1, 1), jnp.float32),
    )
    col = pl.BlockSpec((TB, 1), lambda i: (i, 0))
    return pl.pallas_call(
        _gate_body,
        grid=(NB,),
        in_specs=[
            pl.BlockSpec((TB, D), lambda i: (i, 0)),
            pl.BlockSpec((D, E), lambda i: (0, 0)),
        ],
        out_specs=(col, col, col, col, col, col,
                   pl.BlockSpec((1, 1), lambda i: (0, 0))),
        out_shape=out_shapes,
        scratch_shapes=[
            pltpu.VMEM((1, E), jnp.float32),
            pltpu.VMEM((1, E), jnp.float32),
        ],
    )(x, Wg)


def _ffn_body(ein_ref, w1_ref, b1_ref, w2_ref, b2_ref, out_ref):
    a = ein_ref[0]
    h = jnp.maximum(
        jnp.dot(a, w1_ref[0], preferred_element_type=jnp.float32) + b1_ref[0],
        0.0)
    out_ref[0] = (jnp.dot(h, w2_ref[0], preferred_element_type=jnp.float32)
                  + b2_ref[0])


def _ffn(ein, W1, b1, W2, b2):
    return pl.pallas_call(
        _ffn_body,
        grid=(E, CAP // MB),
        in_specs=[
            pl.BlockSpec((1, MB, D), lambda e, m: (e, m, 0)),
            pl.BlockSpec((1, D, DFF), lambda e, m: (e, 0, 0)),
            pl.BlockSpec((1, 1, DFF), lambda e, m: (e, 0, 0)),
            pl.BlockSpec((1, DFF, OUT), lambda e, m: (e, 0, 0)),
            pl.BlockSpec((1, 1, OUT), lambda e, m: (e, 0, 0)),
        ],
        out_specs=pl.BlockSpec((1, MB, OUT), lambda e, m: (e, m, 0)),
        out_shape=jax.ShapeDtypeStruct((E, CAP, OUT), jnp.float32),
    )(ein, W1, b1, W2, b2)


def kernel(x, Wg, W1, b1, W2, b2):
    s0, s1, v0, v1, g0, g1, aux = _gating(x, Wg)
    s0 = s0.reshape(T)
    s1 = s1.reshape(T)
    v0 = v0.reshape(T)
    v1 = v1.reshape(T)
    g0 = g0.reshape(T)
    g1 = g1.reshape(T)

    inv = jnp.zeros((E * CAP,), jnp.int32)
    inv = inv.at[jnp.where(v0 >= 0, s0, E * CAP)].set(
        jnp.maximum(v0, 0), mode='drop', unique_indices=True)
    inv = inv.at[jnp.where(v1 >= 0, s1, E * CAP)].set(
        jnp.maximum(v1, 0), mode='drop', unique_indices=True)
    ein = x[inv].reshape(E, CAP, D)

    eo = _ffn(ein, W1, b1.reshape(E, 1, DFF), W2, b2.reshape(E, 1, OUT))
    eo = eo.reshape(E * CAP, OUT)

    out = g0[:, None] * eo[s0] + g1[:, None] * eo[s1]
    return out, aux.reshape(())
